# Initial kernel scaffold; baseline (speedup 1.0000x reference)
import jax
import jax.numpy as jnp
from jax.experimental import pallas as pl


def kernel(inputs, pos_embedding):
    raise NotImplementedError("write your pallas kernel here")


# speedup vs baseline: 3.4470x; 3.4470x over previous
import jax
import jax.numpy as jnp
from jax.experimental import pallas as pl


_ROWS_PER_BLOCK = 512


def _copy_body(table_ref, out_ref):
    out_ref[...] = table_ref[...][None]


def kernel(inputs, pos_embedding):
    batch, seq_len = inputs.shape
    max_pos, d_model = pos_embedding.shape
    assert seq_len == max_pos
    rb = _ROWS_PER_BLOCK
    grid = (max_pos // rb, batch)
    out = pl.pallas_call(
        _copy_body,
        grid=grid,
        in_specs=[pl.BlockSpec((rb, d_model), lambda i, j: (i, 0))],
        out_specs=pl.BlockSpec((1, rb, d_model), lambda i, j: (j, i, 0)),
        out_shape=jax.ShapeDtypeStruct((batch, seq_len, d_model), pos_embedding.dtype),
    )(pos_embedding)
    return out


# speedup vs baseline: 4.4560x; 1.2927x over previous
import jax
import jax.numpy as jnp
from jax.experimental import pallas as pl


_ROWS_PER_BLOCK = 1024


def _copy_body(table_ref, out_ref):
    out_ref[...] = table_ref[...][None]


def kernel(inputs, pos_embedding):
    batch, seq_len = inputs.shape
    max_pos, d_model = pos_embedding.shape
    assert seq_len == max_pos
    rb = _ROWS_PER_BLOCK
    grid = (max_pos // rb, batch)
    out = pl.pallas_call(
        _copy_body,
        grid=grid,
        in_specs=[pl.BlockSpec((rb, d_model), lambda i, j: (i, 0))],
        out_specs=pl.BlockSpec((1, rb, d_model), lambda i, j: (j, i, 0)),
        out_shape=jax.ShapeDtypeStruct((batch, seq_len, d_model), pos_embedding.dtype),
    )(pos_embedding)
    return out


# speedup vs baseline: 5.0512x; 1.1336x over previous
import jax
import jax.numpy as jnp
from jax.experimental import pallas as pl


_ROWS_PER_BLOCK = 2048


def _copy_body(table_ref, out_ref):
    out_ref[...] = table_ref[...][None]


def kernel(inputs, pos_embedding):
    batch, seq_len = inputs.shape
    max_pos, d_model = pos_embedding.shape
    assert seq_len == max_pos
    rb = _ROWS_PER_BLOCK
    grid = (max_pos // rb, batch)
    out = pl.pallas_call(
        _copy_body,
        grid=grid,
        in_specs=[pl.BlockSpec((rb, d_model), lambda i, j: (i, 0))],
        out_specs=pl.BlockSpec((1, rb, d_model), lambda i, j: (j, i, 0)),
        out_shape=jax.ShapeDtypeStruct((batch, seq_len, d_model), pos_embedding.dtype),
    )(pos_embedding)
    return out


# speedup vs baseline: 5.3790x; 1.0649x over previous
import jax
import jax.numpy as jnp
from jax.experimental import pallas as pl


_ROWS_PER_BLOCK = 4096


def _copy_body(table_ref, out_ref):
    out_ref[...] = table_ref[...][None]


def kernel(inputs, pos_embedding):
    batch, seq_len = inputs.shape
    max_pos, d_model = pos_embedding.shape
    assert seq_len == max_pos
    rb = _ROWS_PER_BLOCK
    grid = (max_pos // rb, batch)
    out = pl.pallas_call(
        _copy_body,
        grid=grid,
        in_specs=[pl.BlockSpec((rb, d_model), lambda i, j: (i, 0))],
        out_specs=pl.BlockSpec((1, rb, d_model), lambda i, j: (j, i, 0)),
        out_shape=jax.ShapeDtypeStruct((batch, seq_len, d_model), pos_embedding.dtype),
    )(pos_embedding)
    return out
